# Initial kernel scaffold; baseline (speedup 1.0000x reference)
import functools

import jax
import jax.numpy as jnp
from jax import lax
from jax.experimental import pallas as pl
from jax.experimental.pallas import tpu as pltpu
from jax.experimental.pallas import tpu_sc as plsc

N_NODES = 10000
D_FEAT = 128
HID = 128
N_CLS = 64
N_EDGES = 320000

NC = 2
NS = 16
NW = NC * NS
EDGES_PER_TILE = N_EDGES // NW
CHUNK = 80
NCHUNKS = EDGES_PER_TILE // CHUNK
ROWS_PER_TILE = N_NODES // NS
ZROWS = 125
DEG_W = 16

_mesh = plsc.VectorSubcoreMesh(core_axis_name="c", subcore_axis_name="s")


def _zero_fill(buf, nrows, width):
    def row(i, _):
        def col(j, _):
            buf[i, pl.ds(j * 16, 16)] = jnp.zeros((16,), jnp.float32)
            return 0
        return lax.fori_loop(0, width // 16, col, 0)
    lax.fori_loop(0, nrows, row, 0)


@functools.partial(
    pl.kernel,
    out_type=(jax.ShapeDtypeStruct((N_NODES, DEG_W), jnp.float32),
              jax.ShapeDtypeStruct((N_NODES, DEG_W), jnp.float32)),
    mesh=_mesh,
    scratch_types=[
        pltpu.VMEM((CHUNK,), jnp.int32),
        pltpu.VMEM((CHUNK, DEG_W), jnp.float32),
        pltpu.VMEM((ZROWS, DEG_W), jnp.float32),
        pltpu.VMEM_SHARED((N_NODES, DEG_W), jnp.float32),
    ],
)
def _deg_sc(dst_hbm, out0, out1, idx_v, ones_v, zer_v, acc):
    c = lax.axis_index("c")
    s = lax.axis_index("s")
    wid = s * NC + c

    def fill_ones(i, _):
        ones_v[i, :] = jnp.full((DEG_W,), 1.0, jnp.float32)
        return 0
    lax.fori_loop(0, CHUNK, fill_ones, 0)
    _zero_fill(zer_v, ZROWS, DEG_W)
    my_rows = s * ROWS_PER_TILE
    for k in range(ROWS_PER_TILE // ZROWS):
        pltpu.sync_copy(zer_v, acc.at[pl.ds(my_rows + k * ZROWS, ZROWS)])
    plsc.subcore_barrier()

    base = wid * EDGES_PER_TILE

    def body(i, _):
        pltpu.sync_copy(dst_hbm.at[pl.ds(base + i * CHUNK, CHUNK)], idx_v)
        pltpu.sync_copy(ones_v, acc.at[idx_v], add=True)
        return 0
    lax.fori_loop(0, NCHUNKS, body, 0)
    plsc.subcore_barrier()

    @pl.when(c == 0)
    def _():
        pltpu.sync_copy(acc.at[pl.ds(my_rows, ROWS_PER_TILE)],
                        out0.at[pl.ds(my_rows, ROWS_PER_TILE)])

    @pl.when(c == 1)
    def _():
        pltpu.sync_copy(acc.at[pl.ds(my_rows, ROWS_PER_TILE)],
                        out1.at[pl.ds(my_rows, ROWS_PER_TILE)])


@functools.partial(
    pl.kernel,
    out_type=(jax.ShapeDtypeStruct((N_NODES, D_FEAT), jnp.float32),
              jax.ShapeDtypeStruct((N_NODES, D_FEAT), jnp.float32)),
    mesh=_mesh,
    scratch_types=[
        pltpu.VMEM((CHUNK,), jnp.int32),
        pltpu.VMEM((CHUNK,), jnp.int32),
        pltpu.VMEM((CHUNK, D_FEAT), jnp.float32),
        pltpu.VMEM((ZROWS, D_FEAT), jnp.float32),
        pltpu.VMEM_SHARED((N_NODES, D_FEAT), jnp.float32),
        pltpu.SemaphoreType.DMA,
    ],
)
def _agg_sc(y_hbm, src_hbm, dst_hbm, out0, out1,
            si_v, di_v, rows_v, zer_v, acc, sem):
    c = lax.axis_index("c")
    s = lax.axis_index("s")
    wid = s * NC + c

    _zero_fill(zer_v, ZROWS, D_FEAT)
    my_rows = s * ROWS_PER_TILE
    for k in range(ROWS_PER_TILE // ZROWS):
        pltpu.sync_copy(zer_v, acc.at[pl.ds(my_rows + k * ZROWS, ZROWS)])
    plsc.subcore_barrier()

    base = wid * EDGES_PER_TILE

    def body(i, _):
        e0 = base + i * CHUNK
        pltpu.sync_copy(src_hbm.at[pl.ds(e0, CHUNK)], si_v)
        pltpu.sync_copy(dst_hbm.at[pl.ds(e0, CHUNK)], di_v)
        pltpu.async_copy(y_hbm.at[si_v], rows_v, sem).wait()
        pltpu.sync_copy(rows_v, acc.at[di_v], add=True)
        return 0
    lax.fori_loop(0, NCHUNKS, body, 0)
    plsc.subcore_barrier()

    @pl.when(c == 0)
    def _():
        pltpu.sync_copy(acc.at[pl.ds(my_rows, ROWS_PER_TILE)],
                        out0.at[pl.ds(my_rows, ROWS_PER_TILE)])

    @pl.when(c == 1)
    def _():
        pltpu.sync_copy(acc.at[pl.ds(my_rows, ROWS_PER_TILE)],
                        out1.at[pl.ds(my_rows, ROWS_PER_TILE)])


_RB = 1000


def _dinv_of(d0, d1):
    deg = d0[:, 0] + d1[:, 0] + 1.0
    return lax.rsqrt(deg)


def _t1_body(x_ref, w_ref, d0_ref, d1_ref, xw_ref, y_ref):
    xw = jnp.dot(x_ref[...], w_ref[...], preferred_element_type=jnp.float32)
    dinv = _dinv_of(d0_ref[...], d1_ref[...])
    xw_ref[...] = xw
    y_ref[...] = xw * dinv[:, None]


def _t2_body(a0_ref, a1_ref, xw1_ref, d0_ref, d1_ref, w2_ref, b1_ref,
             h1_ref, xw2_ref, y2_ref):
    dinv = _dinv_of(d0_ref[...], d1_ref[...])
    agg = a0_ref[...] + a1_ref[...]
    h1 = jnp.maximum(
        agg * dinv[:, None] + xw1_ref[...] * (dinv * dinv)[:, None]
        + b1_ref[...], 0.0)
    h1_ref[...] = h1
    xw2 = jnp.dot(h1, w2_ref[...], preferred_element_type=jnp.float32)
    xw2_ref[...] = xw2
    y2_ref[...] = xw2 * dinv[:, None]


def _t3_body(a0_ref, a1_ref, xw2_ref, d0_ref, d1_ref, h1_ref,
             wo1_ref, wo2_ref, b2_ref, bo_ref, out_ref):
    dinv = _dinv_of(d0_ref[...], d1_ref[...])
    agg = a0_ref[...] + a1_ref[...]
    h2 = jnp.maximum(
        agg * dinv[:, None] + xw2_ref[...] * (dinv * dinv)[:, None]
        + b2_ref[...], 0.0)
    logits = (jnp.dot(h1_ref[...], wo1_ref[...],
                      preferred_element_type=jnp.float32)
              + jnp.dot(h2, wo2_ref[...],
                        preferred_element_type=jnp.float32)
              + bo_ref[...])
    m = jnp.max(logits, axis=1, keepdims=True)
    lse = m + jnp.log(jnp.sum(jnp.exp(logits - m), axis=1, keepdims=True))
    out_ref[...] = logits - lse


def _row_spec(width):
    return pl.BlockSpec((_RB, width), lambda i: (i, 0))


def _full_spec(shape):
    return pl.BlockSpec(shape, lambda i: (0,) * len(shape))


_GRID = N_NODES // _RB


def _t1(x, w1, d0, d1):
    return pl.pallas_call(
        _t1_body,
        grid=(_GRID,),
        in_specs=[_row_spec(D_FEAT), _full_spec((D_FEAT, HID)),
                  _row_spec(DEG_W), _row_spec(DEG_W)],
        out_specs=[_row_spec(HID), _row_spec(HID)],
        out_shape=[jax.ShapeDtypeStruct((N_NODES, HID), jnp.float32)] * 2,
    )(x, w1, d0, d1)


def _t2(a0, a1, xw1, d0, d1, w2, b1):
    return pl.pallas_call(
        _t2_body,
        grid=(_GRID,),
        in_specs=[_row_spec(HID), _row_spec(HID), _row_spec(HID),
                  _row_spec(DEG_W), _row_spec(DEG_W),
                  _full_spec((HID, HID)), _full_spec((1, HID))],
        out_specs=[_row_spec(HID)] * 3,
        out_shape=[jax.ShapeDtypeStruct((N_NODES, HID), jnp.float32)] * 3,
    )(a0, a1, xw1, d0, d1, w2, b1)


def _t3(a0, a1, xw2, d0, d1, h1, wo1, wo2, b2, bo):
    return pl.pallas_call(
        _t3_body,
        grid=(_GRID,),
        in_specs=[_row_spec(HID), _row_spec(HID), _row_spec(HID),
                  _row_spec(DEG_W), _row_spec(DEG_W), _row_spec(HID),
                  _full_spec((HID, N_CLS)), _full_spec((HID, N_CLS)),
                  _full_spec((1, HID)), _full_spec((1, N_CLS))],
        out_specs=[_row_spec(N_CLS)],
        out_shape=[jax.ShapeDtypeStruct((N_NODES, N_CLS), jnp.float32)],
    )(a0, a1, xw2, d0, d1, h1, wo1, wo2, b2, bo)[0]


def kernel(x, edge_index, W1, b1, W2, b2, Wo, bo):
    ei = edge_index.astype(jnp.int32)
    src = ei[0]
    dst = ei[1]

    d0, d1 = _deg_sc(dst)
    xw1, y1 = _t1(x, W1, d0, d1)
    a0, a1 = _agg_sc(y1, src, dst)
    h1, xw2, y2 = _t2(a0, a1, xw1, d0, d1, W2, b1.reshape(1, HID))
    g0, g1 = _agg_sc(y2, src, dst)
    out = _t3(g0, g1, xw2, d0, d1, h1,
              Wo[:HID], Wo[HID:], b2.reshape(1, HID), bo.reshape(1, N_CLS))
    return out


# speedup vs baseline: 7.3658x; 7.3658x over previous
import jax
import jax.numpy as jnp
from jax import lax
from jax.experimental import pallas as pl
from jax.experimental.pallas import tpu as pltpu
from jax.experimental.pallas import tpu_sc as plsc

N_NODES = 10000
D_FEAT = 128
HID = 128
N_CLS = 64
N_EDGES = 320000

NC = 2
NS = 16
NW = NC * NS
L = 16

CHUNK = 80

EDGES_PER_TILE = N_EDGES // NS
NCHUNKS = EDGES_PER_TILE // CHUNK
NODES_PER_SC = N_NODES // NC
DUMP_ROWS = NS
ACC_ROWS = NODES_PER_SC + DUMP_ROWS
ROWS_MAIN = 312
REM_ROWS = NODES_PER_SC - NS * ROWS_MAIN
REM_BASE = NS * ROWS_MAIN

DEG_EDGES_PER_TILE = N_EDGES // NW
DEG_NCHUNKS = DEG_EDGES_PER_TILE // CHUNK
DEG_W = 16
DROWS_MAIN = 624
DREM_ROWS = N_NODES - NS * DROWS_MAIN
DREM_BASE = NS * DROWS_MAIN

_mesh = plsc.VectorSubcoreMesh(core_axis_name="c", subcore_axis_name="s")


def _zero_fill(buf, nrows, width):
    def row(i, _):
        def col(j, _):
            buf[i, pl.ds(j * L, L)] = jnp.zeros((L,), jnp.float32)
            return 0
        return lax.fori_loop(0, width // L, col, 0)
    lax.fori_loop(0, nrows, row, 0)


def _set_idx(idx_v, base, off, lim):
    iota = lax.iota(jnp.int32, L)

    def one(k, _):
        j = off + k * L
        idx_v[pl.ds(k * L, L)] = base + jnp.minimum(j + iota, lim)
        return 0
    lax.fori_loop(0, CHUNK // L, one, 0)


def _deg_sc_fn(dst_hbm, out0, out1, idx_v, ones_v, zer_v, acc, sem):
    c = lax.axis_index("c")
    s = lax.axis_index("s")
    wid = s * NC + c

    _zero_fill(zer_v, CHUNK, DEG_W)

    def fill_ones(i, _):
        ones_v[i, :] = jnp.full((DEG_W,), 1.0, jnp.float32)
        return 0
    lax.fori_loop(0, CHUNK, fill_ones, 0)

    my_rows = s * DROWS_MAIN

    def zblk(b, _):
        _set_idx(idx_v, my_rows, b * CHUNK, DROWS_MAIN - 1)
        pltpu.sync_copy(zer_v, acc.at[idx_v])
        return 0
    lax.fori_loop(0, (DROWS_MAIN + CHUNK - 1) // CHUNK, zblk, 0)

    @pl.when(s == 0)
    def _():
        _set_idx(idx_v, DREM_BASE, 0, DREM_ROWS - 1)
        pltpu.sync_copy(zer_v, acc.at[idx_v])
    plsc.subcore_barrier()

    base = wid * DEG_EDGES_PER_TILE

    def body(i, _):
        pltpu.sync_copy(dst_hbm.at[pl.ds(base + i * CHUNK, CHUNK)], idx_v)
        pltpu.sync_copy(ones_v, acc.at[idx_v], add=True)
        return 0
    lax.fori_loop(0, DEG_NCHUNKS, body, 0)
    plsc.subcore_barrier()

    def copy_out(dst):
        nfull = DROWS_MAIN // CHUNK
        tail = DROWS_MAIN - nfull * CHUNK
        for b in range(nfull):
            _set_idx(idx_v, my_rows, b * CHUNK, DROWS_MAIN - 1)
            pltpu.async_copy(acc.at[idx_v], zer_v, sem).wait()
            pltpu.sync_copy(zer_v, dst.at[pl.ds(my_rows + b * CHUNK, CHUNK)])
        _set_idx(idx_v, my_rows, nfull * CHUNK, DROWS_MAIN - 1)
        pltpu.async_copy(acc.at[idx_v], zer_v, sem).wait()
        pltpu.sync_copy(zer_v.at[pl.ds(0, tail)],
                        dst.at[pl.ds(my_rows + nfull * CHUNK, tail)])

        @pl.when(s == 0)
        def _():
            _set_idx(idx_v, DREM_BASE, 0, DREM_ROWS - 1)
            pltpu.async_copy(acc.at[idx_v], zer_v, sem).wait()
            pltpu.sync_copy(zer_v.at[pl.ds(0, DREM_ROWS)],
                            dst.at[pl.ds(DREM_BASE, DREM_ROWS)])

    @pl.when(c == 0)
    def _():
        copy_out(out0)

    @pl.when(c == 1)
    def _():
        copy_out(out1)


_deg_sc = pl.kernel(
    _deg_sc_fn,
    out_type=(jax.ShapeDtypeStruct((N_NODES, DEG_W), jnp.float32),
              jax.ShapeDtypeStruct((N_NODES, DEG_W), jnp.float32)),
    mesh=_mesh,
    scratch_types=[
        pltpu.VMEM((CHUNK,), jnp.int32),
        pltpu.VMEM((CHUNK, DEG_W), jnp.float32),
        pltpu.VMEM((CHUNK, DEG_W), jnp.float32),
        pltpu.VMEM_SHARED((N_NODES, DEG_W), jnp.float32),
        pltpu.SemaphoreType.DMA,
    ],
)


def _agg_sc_fn(y_hbm, src_hbm, dst_hbm, out,
               si_v, di_v, rows_v, zer_v, acc, sem):
    c = lax.axis_index("c")
    s = lax.axis_index("s")

    _zero_fill(zer_v, CHUNK, D_FEAT)
    my_rows = s * ROWS_MAIN
    lo = c * NODES_PER_SC
    dump = NODES_PER_SC + s

    def zblk(b, _):
        _set_idx(di_v, my_rows, b * CHUNK, ROWS_MAIN - 1)
        pltpu.sync_copy(zer_v, acc.at[di_v])
        return 0
    lax.fori_loop(0, (ROWS_MAIN + CHUNK - 1) // CHUNK, zblk, 0)

    @pl.when(s == 0)
    def _():
        _set_idx(di_v, REM_BASE, 0, REM_ROWS + DUMP_ROWS - 1)
        pltpu.sync_copy(zer_v, acc.at[di_v])
    plsc.subcore_barrier()

    base = s * EDGES_PER_TILE

    def body(i, _):
        e0 = base + i * CHUNK
        pltpu.sync_copy(src_hbm.at[pl.ds(e0, CHUNK)], si_v)
        pltpu.sync_copy(dst_hbm.at[pl.ds(e0, CHUNK)], di_v)
        pltpu.async_copy(y_hbm.at[si_v], rows_v, sem).wait()
        for k in range(CHUNK // L):
            d = di_v[pl.ds(k * L, L)]
            local = d - lo
            ok = (local >= 0) & (local < NODES_PER_SC)
            di_v[pl.ds(k * L, L)] = jnp.where(ok, local, dump)
        pltpu.sync_copy(rows_v, acc.at[di_v], add=True)
        return 0
    lax.fori_loop(0, NCHUNKS, body, 0)
    plsc.subcore_barrier()

    nfull = ROWS_MAIN // CHUNK
    tail = ROWS_MAIN - nfull * CHUNK
    for b in range(nfull):
        _set_idx(di_v, my_rows, b * CHUNK, ROWS_MAIN - 1)
        pltpu.async_copy(acc.at[di_v], rows_v, sem).wait()
        pltpu.sync_copy(rows_v, out.at[pl.ds(lo + my_rows + b * CHUNK, CHUNK)])
    _set_idx(di_v, my_rows, nfull * CHUNK, ROWS_MAIN - 1)
    pltpu.async_copy(acc.at[di_v], rows_v, sem).wait()
    pltpu.sync_copy(rows_v.at[pl.ds(0, tail)],
                    out.at[pl.ds(lo + my_rows + nfull * CHUNK, tail)])

    @pl.when(s == 0)
    def _():
        _set_idx(di_v, REM_BASE, 0, REM_ROWS - 1)
        pltpu.async_copy(acc.at[di_v], rows_v, sem).wait()
        pltpu.sync_copy(rows_v.at[pl.ds(0, REM_ROWS)],
                        out.at[pl.ds(lo + REM_BASE, REM_ROWS)])


_agg_sc = pl.kernel(
    _agg_sc_fn,
    out_type=jax.ShapeDtypeStruct((N_NODES, D_FEAT), jnp.float32),
    mesh=_mesh,
    scratch_types=[
        pltpu.VMEM((CHUNK,), jnp.int32),
        pltpu.VMEM((CHUNK,), jnp.int32),
        pltpu.VMEM((CHUNK, D_FEAT), jnp.float32),
        pltpu.VMEM((CHUNK, D_FEAT), jnp.float32),
        pltpu.VMEM_SHARED((ACC_ROWS, D_FEAT), jnp.float32),
        pltpu.SemaphoreType.DMA,
    ],
)


_RB = 1000


def _dinv_of(dsum):
    return lax.rsqrt(dsum[:, 0])


def _t1_body(x_ref, w_ref, xw_ref):
    xw_ref[...] = jnp.dot(x_ref[...], w_ref[...],
                          preferred_element_type=jnp.float32)


def _t2_body(a_ref, xw1_ref, ds_ref, w2_ref, b1_ref,
             h1_ref, xw2_ref):
    dinv = _dinv_of(ds_ref[...])
    h1 = jnp.maximum(
        a_ref[...] * dinv[:, None] + xw1_ref[...] * (dinv * dinv)[:, None]
        + b1_ref[...], 0.0)
    h1_ref[...] = h1
    xw2_ref[...] = jnp.dot(h1, w2_ref[...],
                           preferred_element_type=jnp.float32)


def _t3_body(g_ref, xw2_ref, ds_ref, h1_ref,
             wo1_ref, wo2_ref, b2_ref, bo_ref, out_ref):
    dinv = _dinv_of(ds_ref[...])
    h2 = jnp.maximum(
        g_ref[...] * dinv[:, None] + xw2_ref[...] * (dinv * dinv)[:, None]
        + b2_ref[...], 0.0)
    logits = (jnp.dot(h1_ref[...], wo1_ref[...],
                      preferred_element_type=jnp.float32)
              + jnp.dot(h2, wo2_ref[...],
                        preferred_element_type=jnp.float32)
              + bo_ref[...])
    m = jnp.max(logits, axis=1, keepdims=True)
    lse = m + jnp.log(jnp.sum(jnp.exp(logits - m), axis=1, keepdims=True))
    out_ref[...] = logits - lse


def _row_spec(width):
    return pl.BlockSpec((_RB, width), lambda i: (i, 0))


def _full_spec(shape):
    return pl.BlockSpec(shape, lambda i: (0,) * len(shape))


_GRID = N_NODES // _RB


def _t1(x, w1):
    return pl.pallas_call(
        _t1_body,
        grid=(_GRID,),
        in_specs=[_row_spec(D_FEAT), _full_spec((D_FEAT, HID))],
        out_specs=[_row_spec(HID)],
        out_shape=[jax.ShapeDtypeStruct((N_NODES, HID), jnp.float32)],
    )(x, w1)[0]


def _t2(a, xw1, dsum, w2, b1):
    return pl.pallas_call(
        _t2_body,
        grid=(_GRID,),
        in_specs=[_row_spec(HID), _row_spec(HID), _row_spec(DEG_W),
                  _full_spec((HID, HID)), _full_spec((1, HID))],
        out_specs=[_row_spec(HID)] * 2,
        out_shape=[jax.ShapeDtypeStruct((N_NODES, HID), jnp.float32)] * 2,
    )(a, xw1, dsum, w2, b1)


def _t3(g, xw2, dsum, h1, wo1, wo2, b2, bo):
    return pl.pallas_call(
        _t3_body,
        grid=(_GRID,),
        in_specs=[_row_spec(HID), _row_spec(HID),
                  _row_spec(DEG_W), _row_spec(HID),
                  _full_spec((HID, N_CLS)), _full_spec((HID, N_CLS)),
                  _full_spec((1, HID)), _full_spec((1, N_CLS))],
        out_specs=[_row_spec(N_CLS)],
        out_shape=[jax.ShapeDtypeStruct((N_NODES, N_CLS), jnp.float32)],
    )(g, xw2, dsum, h1, wo1, wo2, b2, bo)[0]


def kernel(x, edge_index, W1, b1, W2, b2, Wo, bo):
    ei = edge_index.astype(jnp.int32)
    src = ei[0]
    dst = ei[1]

    d0, d1 = _deg_sc(dst)
    dsum = d0 + d1 + 1.0
    dinv = lax.rsqrt(dsum[:, 0])
    xw1 = _t1(x, W1)
    a = _agg_sc(xw1 * dinv[:, None], src, dst)
    h1, xw2 = _t2(a, xw1, dsum, W2, b1.reshape(1, HID))
    g = _agg_sc(xw2 * dinv[:, None], src, dst)
    out = _t3(g, xw2, dsum, h1,
              Wo[:HID], Wo[HID:], b2.reshape(1, HID), bo.reshape(1, N_CLS))
    return out


# speedup vs baseline: 10.0010x; 1.3578x over previous
import jax
import jax.numpy as jnp
from jax import lax
from jax.experimental import pallas as pl
from jax.experimental.pallas import tpu as pltpu
from jax.experimental.pallas import tpu_sc as plsc

N_NODES = 10000
D_FEAT = 128
HID = 128
N_CLS = 64
N_EDGES = 320000

NC = 2
NS = 16
NW = NC * NS
L = 16

CHUNK = 80

EDGES_PER_TILE = N_EDGES // NS
NCHUNKS = EDGES_PER_TILE // CHUNK
NODES_PER_SC = N_NODES // NC
DUMP_ROWS = NS
ACC_ROWS = NODES_PER_SC + DUMP_ROWS
ROWS_MAIN = 312
REM_ROWS = NODES_PER_SC - NS * ROWS_MAIN
REM_BASE = NS * ROWS_MAIN

DEG_EDGES_PER_TILE = N_EDGES // NW
DEG_NCHUNKS = DEG_EDGES_PER_TILE // CHUNK
DEG_W = 16
DROWS_MAIN = 624
DREM_ROWS = N_NODES - NS * DROWS_MAIN
DREM_BASE = NS * DROWS_MAIN

_mesh = plsc.VectorSubcoreMesh(core_axis_name="c", subcore_axis_name="s")


def _zero_fill(buf, nrows, width):
    def row(i, _):
        def col(j, _):
            buf[i, pl.ds(j * L, L)] = jnp.zeros((L,), jnp.float32)
            return 0
        return lax.fori_loop(0, width // L, col, 0)
    lax.fori_loop(0, nrows, row, 0)


def _set_idx(idx_v, base, off, lim):
    iota = lax.iota(jnp.int32, L)

    def one(k, _):
        j = off + k * L
        idx_v[pl.ds(k * L, L)] = base + jnp.minimum(j + iota, lim)
        return 0
    lax.fori_loop(0, CHUNK // L, one, 0)


def _deg_sc_fn(dst_hbm, out0, out1, idx_v, ones_v, zer_v, acc, sem):
    c = lax.axis_index("c")
    s = lax.axis_index("s")
    wid = s * NC + c

    _zero_fill(zer_v, CHUNK, DEG_W)

    def fill_ones(i, _):
        ones_v[i, :] = jnp.full((DEG_W,), 1.0, jnp.float32)
        return 0
    lax.fori_loop(0, CHUNK, fill_ones, 0)

    my_rows = s * DROWS_MAIN

    def zblk(b, _):
        _set_idx(idx_v, my_rows, b * CHUNK, DROWS_MAIN - 1)
        pltpu.sync_copy(zer_v, acc.at[idx_v])
        return 0
    lax.fori_loop(0, (DROWS_MAIN + CHUNK - 1) // CHUNK, zblk, 0)

    @pl.when(s == 0)
    def _():
        _set_idx(idx_v, DREM_BASE, 0, DREM_ROWS - 1)
        pltpu.sync_copy(zer_v, acc.at[idx_v])
    plsc.subcore_barrier()

    base = wid * DEG_EDGES_PER_TILE

    def body(i, _):
        pltpu.sync_copy(dst_hbm.at[pl.ds(base + i * CHUNK, CHUNK)], idx_v)
        pltpu.sync_copy(ones_v, acc.at[idx_v], add=True)
        return 0
    lax.fori_loop(0, DEG_NCHUNKS, body, 0)
    plsc.subcore_barrier()

    def copy_out(dst):
        nfull = DROWS_MAIN // CHUNK
        tail = DROWS_MAIN - nfull * CHUNK
        for b in range(nfull):
            _set_idx(idx_v, my_rows, b * CHUNK, DROWS_MAIN - 1)
            pltpu.async_copy(acc.at[idx_v], zer_v, sem).wait()
            pltpu.sync_copy(zer_v, dst.at[pl.ds(my_rows + b * CHUNK, CHUNK)])
        _set_idx(idx_v, my_rows, nfull * CHUNK, DROWS_MAIN - 1)
        pltpu.async_copy(acc.at[idx_v], zer_v, sem).wait()
        pltpu.sync_copy(zer_v.at[pl.ds(0, tail)],
                        dst.at[pl.ds(my_rows + nfull * CHUNK, tail)])

        @pl.when(s == 0)
        def _():
            _set_idx(idx_v, DREM_BASE, 0, DREM_ROWS - 1)
            pltpu.async_copy(acc.at[idx_v], zer_v, sem).wait()
            pltpu.sync_copy(zer_v.at[pl.ds(0, DREM_ROWS)],
                            dst.at[pl.ds(DREM_BASE, DREM_ROWS)])

    @pl.when(c == 0)
    def _():
        copy_out(out0)

    @pl.when(c == 1)
    def _():
        copy_out(out1)


_deg_sc = pl.kernel(
    _deg_sc_fn,
    out_type=(jax.ShapeDtypeStruct((N_NODES, DEG_W), jnp.float32),
              jax.ShapeDtypeStruct((N_NODES, DEG_W), jnp.float32)),
    mesh=_mesh,
    scratch_types=[
        pltpu.VMEM((CHUNK,), jnp.int32),
        pltpu.VMEM((CHUNK, DEG_W), jnp.float32),
        pltpu.VMEM((CHUNK, DEG_W), jnp.float32),
        pltpu.VMEM_SHARED((N_NODES, DEG_W), jnp.float32),
        pltpu.SemaphoreType.DMA,
    ],
)


def _agg_sc_fn(y_hbm, src_hbm, dst_hbm, out,
               si_v, di_v, rows_v, si_b, di_b, rows_b, zer_v, acc,
               sem, sem_b):
    c = lax.axis_index("c")
    s = lax.axis_index("s")

    _zero_fill(zer_v, CHUNK, D_FEAT)
    my_rows = s * ROWS_MAIN
    lo = c * NODES_PER_SC
    dump = NODES_PER_SC + s

    def zblk(b, _):
        _set_idx(di_v, my_rows, b * CHUNK, ROWS_MAIN - 1)
        pltpu.sync_copy(zer_v, acc.at[di_v])
        return 0
    lax.fori_loop(0, (ROWS_MAIN + CHUNK - 1) // CHUNK, zblk, 0)

    @pl.when(s == 0)
    def _():
        _set_idx(di_v, REM_BASE, 0, REM_ROWS + DUMP_ROWS - 1)
        pltpu.sync_copy(zer_v, acc.at[di_v])
    plsc.subcore_barrier()

    base = s * EDGES_PER_TILE

    def load_idx(e0, si, di):
        pltpu.sync_copy(src_hbm.at[pl.ds(e0, CHUNK)], si)
        pltpu.sync_copy(dst_hbm.at[pl.ds(e0, CHUNK)], di)

    def redirect(di):
        for k in range(CHUNK // L):
            d = di[pl.ds(k * L, L)]
            local = d - lo
            ok = (local >= 0) & (local < NODES_PER_SC)
            di[pl.ds(k * L, L)] = jnp.where(ok, local, dump)

    npairs = NCHUNKS // 2

    def body(g, _):
        e0 = base + 2 * g * CHUNK
        load_idx(e0, si_v, di_v)
        cp_a = pltpu.async_copy(y_hbm.at[si_v], rows_v, sem)
        load_idx(e0 + CHUNK, si_b, di_b)
        cp_b = pltpu.async_copy(y_hbm.at[si_b], rows_b, sem_b)
        cp_a.wait()
        redirect(di_v)
        pltpu.sync_copy(rows_v, acc.at[di_v], add=True)
        cp_b.wait()
        redirect(di_b)
        pltpu.sync_copy(rows_b, acc.at[di_b], add=True)
        return 0
    lax.fori_loop(0, npairs, body, 0)
    plsc.subcore_barrier()

    nfull = ROWS_MAIN // CHUNK
    tail = ROWS_MAIN - nfull * CHUNK
    for b in range(nfull):
        _set_idx(di_v, my_rows, b * CHUNK, ROWS_MAIN - 1)
        pltpu.async_copy(acc.at[di_v], rows_v, sem).wait()
        pltpu.sync_copy(rows_v, out.at[pl.ds(lo + my_rows + b * CHUNK, CHUNK)])
    _set_idx(di_v, my_rows, nfull * CHUNK, ROWS_MAIN - 1)
    pltpu.async_copy(acc.at[di_v], rows_v, sem).wait()
    pltpu.sync_copy(rows_v.at[pl.ds(0, tail)],
                    out.at[pl.ds(lo + my_rows + nfull * CHUNK, tail)])

    @pl.when(s == 0)
    def _():
        _set_idx(di_v, REM_BASE, 0, REM_ROWS - 1)
        pltpu.async_copy(acc.at[di_v], rows_v, sem).wait()
        pltpu.sync_copy(rows_v.at[pl.ds(0, REM_ROWS)],
                        out.at[pl.ds(lo + REM_BASE, REM_ROWS)])


_agg_sc = pl.kernel(
    _agg_sc_fn,
    out_type=jax.ShapeDtypeStruct((N_NODES, D_FEAT), jnp.float32),
    mesh=_mesh,
    scratch_types=[
        pltpu.VMEM((CHUNK,), jnp.int32),
        pltpu.VMEM((CHUNK,), jnp.int32),
        pltpu.VMEM((CHUNK, D_FEAT), jnp.float32),
        pltpu.VMEM((CHUNK,), jnp.int32),
        pltpu.VMEM((CHUNK,), jnp.int32),
        pltpu.VMEM((CHUNK, D_FEAT), jnp.float32),
        pltpu.VMEM((CHUNK, D_FEAT), jnp.float32),
        pltpu.VMEM_SHARED((ACC_ROWS, D_FEAT), jnp.float32),
        pltpu.SemaphoreType.DMA,
        pltpu.SemaphoreType.DMA,
    ],
)


_RB = 1000


def _dinv_of(dsum):
    return lax.rsqrt(dsum[:, 0])


def _t1_body(x_ref, w_ref, xw_ref):
    xw_ref[...] = jnp.dot(x_ref[...], w_ref[...],
                          preferred_element_type=jnp.float32)


def _t2_body(a_ref, xw1_ref, ds_ref, w2_ref, b1_ref,
             h1_ref, xw2_ref):
    dinv = _dinv_of(ds_ref[...])
    h1 = jnp.maximum(
        a_ref[...] * dinv[:, None] + xw1_ref[...] * (dinv * dinv)[:, None]
        + b1_ref[...], 0.0)
    h1_ref[...] = h1
    xw2_ref[...] = jnp.dot(h1, w2_ref[...],
                           preferred_element_type=jnp.float32)


def _t3_body(g_ref, xw2_ref, ds_ref, h1_ref,
             wo1_ref, wo2_ref, b2_ref, bo_ref, out_ref):
    dinv = _dinv_of(ds_ref[...])
    h2 = jnp.maximum(
        g_ref[...] * dinv[:, None] + xw2_ref[...] * (dinv * dinv)[:, None]
        + b2_ref[...], 0.0)
    logits = (jnp.dot(h1_ref[...], wo1_ref[...],
                      preferred_element_type=jnp.float32)
              + jnp.dot(h2, wo2_ref[...],
                        preferred_element_type=jnp.float32)
              + bo_ref[...])
    m = jnp.max(logits, axis=1, keepdims=True)
    lse = m + jnp.log(jnp.sum(jnp.exp(logits - m), axis=1, keepdims=True))
    out_ref[...] = logits - lse


def _row_spec(width):
    return pl.BlockSpec((_RB, width), lambda i: (i, 0))


def _full_spec(shape):
    return pl.BlockSpec(shape, lambda i: (0,) * len(shape))


_GRID = N_NODES // _RB


def _t1(x, w1):
    return pl.pallas_call(
        _t1_body,
        grid=(_GRID,),
        in_specs=[_row_spec(D_FEAT), _full_spec((D_FEAT, HID))],
        out_specs=[_row_spec(HID)],
        out_shape=[jax.ShapeDtypeStruct((N_NODES, HID), jnp.float32)],
    )(x, w1)[0]


def _t2(a, xw1, dsum, w2, b1):
    return pl.pallas_call(
        _t2_body,
        grid=(_GRID,),
        in_specs=[_row_spec(HID), _row_spec(HID), _row_spec(DEG_W),
                  _full_spec((HID, HID)), _full_spec((1, HID))],
        out_specs=[_row_spec(HID)] * 2,
        out_shape=[jax.ShapeDtypeStruct((N_NODES, HID), jnp.float32)] * 2,
    )(a, xw1, dsum, w2, b1)


def _t3(g, xw2, dsum, h1, wo1, wo2, b2, bo):
    return pl.pallas_call(
        _t3_body,
        grid=(_GRID,),
        in_specs=[_row_spec(HID), _row_spec(HID),
                  _row_spec(DEG_W), _row_spec(HID),
                  _full_spec((HID, N_CLS)), _full_spec((HID, N_CLS)),
                  _full_spec((1, HID)), _full_spec((1, N_CLS))],
        out_specs=[_row_spec(N_CLS)],
        out_shape=[jax.ShapeDtypeStruct((N_NODES, N_CLS), jnp.float32)],
    )(g, xw2, dsum, h1, wo1, wo2, b2, bo)[0]


def kernel(x, edge_index, W1, b1, W2, b2, Wo, bo):
    ei = edge_index.astype(jnp.int32)
    src = ei[0]
    dst = ei[1]

    d0, d1 = _deg_sc(dst)
    dsum = d0 + d1 + 1.0
    dinv = lax.rsqrt(dsum[:, 0])
    xw1 = _t1(x, W1)
    a = _agg_sc(xw1 * dinv[:, None], src, dst)
    h1, xw2 = _t2(a, xw1, dsum, W2, b1.reshape(1, HID))
    g = _agg_sc(xw2 * dinv[:, None], src, dst)
    out = _t3(g, xw2, dsum, h1,
              Wo[:HID], Wo[HID:], b2.reshape(1, HID), bo.reshape(1, N_CLS))
    return out
